# baseline (speedup 1.0000x reference)
import jax
import jax.numpy as jnp
from jax import lax
from jax.experimental import pallas as pl

_N = 256
_D = 128
_MARGIN = 1.0
_BIG = 1e9


def _triplet_kernel(emb_ref, trow_ref, tcol_ref, loss_ref, ratio_ref):
    E = emb_ref[:]
    t_row = trow_ref[:]
    t_col = tcol_ref[:]

    G = lax.dot_general(E, E, (((1,), (1,)), ((), ())),
                        preferred_element_type=jnp.float32)
    EE = E * E
    sq_col = jnp.sum(EE, axis=1, keepdims=True)
    ones_d = jnp.ones((1, _D), jnp.float32)
    sq_row = lax.dot_general(ones_d, EE, (((1,), (1,)), ((), ())),
                             preferred_element_type=jnp.float32)
    Dm = sq_col + sq_row - 2.0 * G

    same = t_col == t_row
    row_i = lax.broadcasted_iota(jnp.int32, (_N, _N), 0)
    col_i = lax.broadcasted_iota(jnp.int32, (_N, _N), 1)
    aptf = jnp.where(same & (row_i > col_i), 1.0, 0.0)
    negf = jnp.where(same, 0.0, 1.0)

    iota_col = lax.broadcasted_iota(jnp.int32, (_N, 1), 0)

    def body(i, carry):
        tot_row, viol_row = carry
        oh = (iota_col == i).astype(jnp.float32)
        d_col = lax.dot_general(Dm, oh, (((1,), (0,)), ((), ())),
                                preferred_element_type=jnp.float32)
        ap_col = lax.dot_general(aptf, oh, (((1,), (0,)), ((), ())),
                                 preferred_element_type=jnp.float32)
        d_row = lax.dot_general(oh, Dm, (((0,), (0,)), ((), ())),
                                preferred_element_type=jnp.float32)
        neg_row = lax.dot_general(oh, negf, (((0,), (0,)), ((), ())),
                                  preferred_element_type=jnp.float32)
        a = jnp.where(ap_col > 0.5, d_col + _MARGIN, -_BIG)
        b = jnp.where(neg_row > 0.5, d_row, _BIG)
        T = a - b
        tot_row = tot_row + jnp.sum(jnp.maximum(T, 0.0), axis=0,
                                    keepdims=True)
        viol_row = viol_row + jnp.sum(jnp.where(T > 0.0, 1.0, 0.0),
                                      axis=0, keepdims=True)
        return tot_row, viol_row

    zeros = jnp.zeros((1, _N), jnp.float32)
    tot_row, viol_row = lax.fori_loop(0, _N, body, (zeros, zeros))

    total = jnp.sum(tot_row)
    viol = jnp.sum(viol_row)

    ones_n = jnp.ones((1, _N), jnp.float32)
    p_row = lax.dot_general(ones_n, aptf, (((1,), (0,)), ((), ())),
                            preferred_element_type=jnp.float32)
    m_row = lax.dot_general(ones_n, negf, (((1,), (0,)), ((), ())),
                            preferred_element_type=jnp.float32)
    count = jnp.sum(p_row * m_row)

    has = count > 0.5
    loss_sum = jnp.where(has, total, jnp.float32(1.0))
    ratio = jnp.where(has, viol / jnp.maximum(count, 1.0),
                      jnp.float32(1.0))
    loss_ref[...] = jnp.broadcast_to(loss_sum, (1, 1))
    ratio_ref[...] = jnp.broadcast_to(ratio, (1, 1))


def kernel(embeddings, target):
    t32 = target.astype(jnp.int32)
    t_row = t32.reshape(1, _N)
    t_col = t32.reshape(_N, 1)
    loss, ratio = pl.pallas_call(
        _triplet_kernel,
        out_shape=(jax.ShapeDtypeStruct((1, 1), jnp.float32),
                   jax.ShapeDtypeStruct((1, 1), jnp.float32)),
    )(embeddings.astype(jnp.float32), t_row, t_col)
    return (loss[0, 0], ratio[0, 0])


# speedup vs baseline: 4.5739x; 4.5739x over previous
import jax
import jax.numpy as jnp
from jax import lax
from jax.experimental import pallas as pl
from jax.experimental.pallas import tpu as pltpu

_N = 256
_D = 128
_MARGIN = 1.0
_BIG = 1e9
_BLK = 8
_NBLK = _N // _BLK


def _triplet_kernel(emb_ref, trow_ref, tcol_ref, loss_ref, ratio_ref,
                    a_s, b_s):
    E = emb_ref[:]
    t_row = trow_ref[:]
    t_col = tcol_ref[:]

    G = lax.dot_general(E, E, (((1,), (1,)), ((), ())),
                        preferred_element_type=jnp.float32)
    EE = E * E
    sq_col = jnp.sum(EE, axis=1, keepdims=True)
    ones_d = jnp.ones((1, _D), jnp.float32)
    sq_row = lax.dot_general(ones_d, EE, (((1,), (1,)), ((), ())),
                             preferred_element_type=jnp.float32)
    Dm = sq_col + sq_row - 2.0 * G

    same = t_col == t_row
    row_i = lax.broadcasted_iota(jnp.int32, (_N, _N), 0)
    col_i = lax.broadcasted_iota(jnp.int32, (_N, _N), 1)
    apf = jnp.where(same & (row_i < col_i), 1.0, 0.0)
    negf = jnp.where(same, 0.0, 1.0)

    a_s[...] = jnp.where(apf > 0.5, Dm + _MARGIN, -_BIG)
    b_s[...] = jnp.where(negf > 0.5, Dm, _BIG)

    def body(bi, carry):
        tot_acc, viol_acc = carry
        i0 = bi * _BLK
        A = a_s[pl.ds(i0, _BLK), :]
        B = b_s[pl.ds(i0, _BLK), :]
        T = A[:, :, None] - B[:, None, :]
        tot_acc = tot_acc + jnp.sum(jnp.maximum(T, 0.0), axis=1)
        viol_acc = viol_acc + jnp.sum(jnp.where(T > 0.0, 1.0, 0.0),
                                      axis=1)
        return tot_acc, viol_acc

    zeros = jnp.zeros((_BLK, _N), jnp.float32)
    tot_acc, viol_acc = lax.fori_loop(0, _NBLK, body, (zeros, zeros))

    total = jnp.sum(tot_acc)
    viol = jnp.sum(viol_acc)

    ones_n = jnp.ones((1, _N), jnp.float32)
    p_row = lax.dot_general(ones_n, apf, (((1,), (1,)), ((), ())),
                            preferred_element_type=jnp.float32)
    m_row = lax.dot_general(ones_n, negf, (((1,), (1,)), ((), ())),
                            preferred_element_type=jnp.float32)
    count = jnp.sum(p_row * m_row)

    has = count > 0.5
    loss_sum = jnp.where(has, total, jnp.float32(1.0))
    ratio = jnp.where(has, viol / jnp.maximum(count, 1.0),
                      jnp.float32(1.0))
    loss_ref[...] = jnp.broadcast_to(loss_sum, (1, 1))
    ratio_ref[...] = jnp.broadcast_to(ratio, (1, 1))


def kernel(embeddings, target):
    t32 = target.astype(jnp.int32)
    t_row = t32.reshape(1, _N)
    t_col = t32.reshape(_N, 1)
    loss, ratio = pl.pallas_call(
        _triplet_kernel,
        out_shape=(jax.ShapeDtypeStruct((1, 1), jnp.float32),
                   jax.ShapeDtypeStruct((1, 1), jnp.float32)),
        scratch_shapes=[pltpu.VMEM((_N, _N), jnp.float32),
                        pltpu.VMEM((_N, _N), jnp.float32)],
    )(embeddings.astype(jnp.float32), t_row, t_col)
    return (loss[0, 0], ratio[0, 0])
